# Initial kernel scaffold; baseline (speedup 1.0000x reference)
import jax
import jax.numpy as jnp
from jax.experimental import pallas as pl


def kernel(x, coords, mask, W1, b1, ln1_g, ln1_b, W2, b2, ln2_g, ln2_b):
    raise NotImplementedError("write your pallas kernel here")


# speedup vs baseline: 5.8423x; 5.8423x over previous
import functools

import jax
import jax.numpy as jnp
from jax.experimental import pallas as pl
from jax.experimental.pallas import tpu as pltpu
from jax.experimental.pallas import tpu_sc as plsc

_F32 = jnp.float32
_HIGH = jax.lax.Precision.HIGHEST


def _dot(a, b, dims):
    return jax.lax.dot_general(a, b, (dims, ((), ())),
                               preferred_element_type=_F32, precision=_HIGH)


def _knn_body(xr_ref, xf_ref, idx_ref, *, k, n):
    xr = xr_ref[0]
    xf = xf_ref[0]
    br = xr.shape[0]
    nr = jnp.maximum(jnp.sqrt(jnp.sum(xr * xr, axis=1, keepdims=True)), 1e-12)
    xrn = xr / nr
    nf = jnp.maximum(jnp.sqrt(jnp.sum(xf * xf, axis=1, keepdims=True)), 1e-12)
    xfn = xf / nf
    s = _dot(xrn, xfn, ((1,), (1,)))
    xx_r = jnp.sum(xrn * xrn, axis=1, keepdims=True)
    xx_f = jnp.sum(xfn * xfn, axis=1)
    row = (2.0 * s - xx_r) - xx_f[None, :]

    iota = jax.lax.broadcasted_iota(jnp.int32, (br, n), 1)
    bigi = jnp.int32(n)
    cols = []
    for _ in range(k):
        m = jnp.max(row, axis=1, keepdims=True)
        a = jnp.min(jnp.where(row == m, iota, bigi), axis=1, keepdims=True)
        cols.append(a)
        row = jnp.where(iota == a, -jnp.inf, row)
    base = (pl.program_id(0) * n).astype(jnp.int32)
    idx_ref[0] = jnp.concatenate(cols, axis=1) + base


def _knn_call(x, br, k):
    b, n, c = x.shape
    grid = (b, n // br)
    return pl.pallas_call(
        functools.partial(_knn_body, k=k, n=n),
        grid=grid,
        in_specs=[
            pl.BlockSpec((1, br, c), lambda bi, i: (bi, i, 0)),
            pl.BlockSpec((1, n, c), lambda bi, i: (bi, 0, 0)),
        ],
        out_specs=pl.BlockSpec((1, br, k), lambda bi, i: (bi, i, 0)),
        out_shape=jax.ShapeDtypeStruct((b, n, k), jnp.int32),
    )(x, x)


def _pq_body(x_ref, wp_ref, wq_ref, b1_ref, p_ref, q_ref):
    x = x_ref[...]
    p_ref[...] = _dot(x, wp_ref[...], ((1,), (0,))) + b1_ref[...]
    q_ref[...] = _dot(x, wq_ref[...], ((1,), (0,)))


def _pq_call(x2d, wp, wq, b1r, br):
    m, c = x2d.shape
    grid = (m // br,)
    return pl.pallas_call(
        _pq_body,
        grid=grid,
        in_specs=[
            pl.BlockSpec((br, c), lambda i: (i, 0)),
            pl.BlockSpec((c, c), lambda i: (0, 0)),
            pl.BlockSpec((c, c), lambda i: (0, 0)),
            pl.BlockSpec((1, c), lambda i: (0, 0)),
        ],
        out_specs=[
            pl.BlockSpec((br, c), lambda i: (i, 0)),
            pl.BlockSpec((br, c), lambda i: (i, 0)),
        ],
        out_shape=[
            jax.ShapeDtypeStruct((m, c), _F32),
            jax.ShapeDtypeStruct((m, c), _F32),
        ],
    )(x2d, wp, wq, b1r)


def _sc_gather_call(table, gidx):
    r, d = table.shape
    (m,) = gidx.shape
    try:
        info = plsc.get_sparse_core_info()
        nc, ns = info.num_cores, info.num_subcores
    except Exception:
        nc, ns = 2, 16
    nw = nc * ns
    ch = 128
    per_w = m // nw
    nchunk = per_w // ch
    mesh = plsc.VectorSubcoreMesh(core_axis_name="c", subcore_axis_name="s")

    @functools.partial(
        pl.kernel,
        out_type=jax.ShapeDtypeStruct((m, d), _F32),
        mesh=mesh,
        scratch_types=[
            pltpu.VMEM((ch,), jnp.int32),
            pltpu.VMEM((ch, d), _F32),
            pltpu.SemaphoreType.DMA,
        ],
    )
    def k(table_hbm, idx_hbm, out_hbm, idx_v, rows_v, sem):
        wid = jax.lax.axis_index("s") * nc + jax.lax.axis_index("c")
        base0 = wid * per_w

        def body(i, carry):
            base = base0 + i * ch
            pltpu.sync_copy(idx_hbm.at[pl.ds(base, ch)], idx_v)
            pltpu.async_copy(table_hbm.at[idx_v], rows_v, sem).wait()
            pltpu.sync_copy(rows_v, out_hbm.at[pl.ds(base, ch)])
            return carry

        jax.lax.fori_loop(0, nchunk, body, 0)

    return k(table, gidx)


def _edge_body(p_ref, qg_ref, g_ref, bb_ref, out_ref):
    p = p_ref[0]
    qg = qg_ref[0]
    h = jnp.maximum(p[:, None, :] + qg, 0.0)
    mu = jnp.mean(h, axis=-1, keepdims=True)
    var = jnp.mean((h - mu) ** 2, axis=-1, keepdims=True)
    ln = (h - mu) / jnp.sqrt(var + 1e-5) * g_ref[0][None, :] + bb_ref[0][None, :]
    out_ref[0] = jnp.max(ln, axis=1)


def _edge_call(p3, qg, ln1_g, ln1_b, br):
    b, n, k, c = qg.shape
    grid = (b, n // br)
    return pl.pallas_call(
        _edge_body,
        grid=grid,
        in_specs=[
            pl.BlockSpec((1, br, c), lambda bi, i: (bi, i, 0)),
            pl.BlockSpec((1, br, k, c), lambda bi, i: (bi, i, 0, 0)),
            pl.BlockSpec((1, c), lambda bi, i: (0, 0)),
            pl.BlockSpec((1, c), lambda bi, i: (0, 0)),
        ],
        out_specs=pl.BlockSpec((1, br, c), lambda bi, i: (bi, i, 0)),
        out_shape=jax.ShapeDtypeStruct((b, n, c), _F32),
    )(p3, qg, ln1_g.reshape(1, c), ln1_b.reshape(1, c))


def _pool_body(loc_ref, m_ref, g_ref):
    loc = loc_ref[0]
    msk = m_ref[0]
    masked = jnp.where(msk == 0.0, -1000000000.0, loc)
    g_ref[0, 0] = jnp.max(masked, axis=0)


def _pool_call(local, mask):
    b, n, c = local.shape
    return pl.pallas_call(
        _pool_body,
        grid=(b,),
        in_specs=[
            pl.BlockSpec((1, n, c), lambda bi: (bi, 0, 0)),
            pl.BlockSpec((1, n, 1), lambda bi: (bi, 0, 0)),
        ],
        out_specs=pl.BlockSpec((1, 1, c), lambda bi: (bi, 0, 0)),
        out_shape=jax.ShapeDtypeStruct((b, 1, c), _F32),
    )(local, mask[:, :, None]).reshape(b, c)


def _head_body(g_ref, w_ref, b2_ref, lg_ref, lb_ref, out_ref):
    o = _dot(g_ref[...], w_ref[...], ((1,), (0,))) + b2_ref[...]
    mu = jnp.mean(o, axis=-1, keepdims=True)
    var = jnp.mean((o - mu) ** 2, axis=-1, keepdims=True)
    out_ref[...] = (o - mu) / jnp.sqrt(var + 1e-5) * lg_ref[...] + lb_ref[...]


def _head_call(g, w2t, b2, ln2_g, ln2_b):
    b, c = g.shape
    return pl.pallas_call(
        _head_body,
        in_specs=[pl.BlockSpec((b, c), lambda: (0, 0)),
                  pl.BlockSpec((c, c), lambda: (0, 0)),
                  pl.BlockSpec((1, c), lambda: (0, 0)),
                  pl.BlockSpec((1, c), lambda: (0, 0)),
                  pl.BlockSpec((1, c), lambda: (0, 0))],
        out_specs=pl.BlockSpec((b, c), lambda: (0, 0)),
        out_shape=jax.ShapeDtypeStruct((b, c), _F32),
    )(g, w2t, b2.reshape(1, c), ln2_g.reshape(1, c), ln2_b.reshape(1, c))


def kernel(x, coords, mask, W1, b1, ln1_g, ln1_b, W2, b2, ln2_g, ln2_b):
    b, n, c = x.shape
    k = 16
    br = 256 if n % 256 == 0 else n

    gidx = _knn_call(x, br, k)

    wq = jnp.transpose(W1[:, c:])
    wp = jnp.transpose(W1[:, :c] - W1[:, c:])
    p2, q2 = _pq_call(x.reshape(b * n, c), wp, wq, b1.reshape(1, c), 512)
    p3 = p2.reshape(b, n, c)

    qg2 = jnp.take(q2, gidx.reshape(-1), axis=0)
    qg = qg2.reshape(b, n, k, c)

    local = _edge_call(p3, qg, ln1_g, ln1_b, br)
    g = _pool_call(local, mask)
    return _head_call(g, jnp.transpose(W2), b2, ln2_g, ln2_b)


# speedup vs baseline: 6.5879x; 1.1276x over previous
import functools

import jax
import jax.numpy as jnp
from jax.experimental import pallas as pl
from jax.experimental.pallas import tpu as pltpu
from jax.experimental.pallas import tpu_sc as plsc

_F32 = jnp.float32
_HIGH = jax.lax.Precision.HIGHEST


def _dot(a, b, dims):
    return jax.lax.dot_general(a, b, (dims, ((), ())),
                               preferred_element_type=_F32, precision=_HIGH)


def _knn_body(xr_ref, xf_ref, idx_ref, *, k, n):
    xr = xr_ref[0]
    xf = xf_ref[0]
    br = xr.shape[0]
    nr = jnp.maximum(jnp.sqrt(jnp.sum(xr * xr, axis=1, keepdims=True)), 1e-12)
    xrn = xr / nr
    nf = jnp.maximum(jnp.sqrt(jnp.sum(xf * xf, axis=1, keepdims=True)), 1e-12)
    xfn = xf / nf
    s = _dot(xrn, xfn, ((1,), (1,)))
    xx_r = jnp.sum(xrn * xrn, axis=1, keepdims=True)
    xx_f = jnp.sum(xfn * xfn, axis=1)
    row = (2.0 * s - xx_r) - xx_f[None, :]

    nlev = 4
    ng = n // 128
    slices = [row[:, g * 128:(g + 1) * 128] for g in range(ng)]
    lvl_v, lvl_i = [], []
    for _ in range(nlev):
        gm = functools.reduce(jnp.maximum, slices)
        wg = functools.reduce(
            jnp.minimum,
            [jnp.where(s == gm, jnp.int32(g), jnp.int32(ng))
             for g, s in enumerate(slices)])
        lvl_v.append(gm)
        lvl_i.append(wg)
        slices = [jnp.where(wg == g, -jnp.inf, s)
                  for g, s in enumerate(slices)]
    jiota = jax.lax.broadcasted_iota(jnp.int32, (br, 128), 1)
    cand = jnp.concatenate(lvl_v, axis=1)
    cidx = jnp.concatenate([w * 128 + jiota for w in lvl_i], axis=1)
    bigi = jnp.int32(n)
    cols = []
    for _ in range(k):
        m = jnp.max(cand, axis=1, keepdims=True)
        a = jnp.min(jnp.where(cand == m, cidx, bigi), axis=1, keepdims=True)
        cols.append(a)
        cand = jnp.where(cidx == a, -jnp.inf, cand)
    base = (pl.program_id(0) * n).astype(jnp.int32)
    idx_ref[0] = jnp.concatenate(cols, axis=1) + base


def _knn_call(x, br, k):
    b, n, c = x.shape
    grid = (b, n // br)
    return pl.pallas_call(
        functools.partial(_knn_body, k=k, n=n),
        grid=grid,
        in_specs=[
            pl.BlockSpec((1, br, c), lambda bi, i: (bi, i, 0)),
            pl.BlockSpec((1, n, c), lambda bi, i: (bi, 0, 0)),
        ],
        out_specs=pl.BlockSpec((1, br, k), lambda bi, i: (bi, i, 0)),
        out_shape=jax.ShapeDtypeStruct((b, n, k), jnp.int32),
    )(x, x)


def _pq_body(x_ref, wp_ref, wq_ref, b1_ref, p_ref, q_ref):
    x = x_ref[...]
    p_ref[...] = _dot(x, wp_ref[...], ((1,), (0,))) + b1_ref[...]
    q_ref[...] = _dot(x, wq_ref[...], ((1,), (0,)))


def _pq_call(x2d, wp, wq, b1r, br):
    m, c = x2d.shape
    grid = (m // br,)
    return pl.pallas_call(
        _pq_body,
        grid=grid,
        in_specs=[
            pl.BlockSpec((br, c), lambda i: (i, 0)),
            pl.BlockSpec((c, c), lambda i: (0, 0)),
            pl.BlockSpec((c, c), lambda i: (0, 0)),
            pl.BlockSpec((1, c), lambda i: (0, 0)),
        ],
        out_specs=[
            pl.BlockSpec((br, c), lambda i: (i, 0)),
            pl.BlockSpec((br, c), lambda i: (i, 0)),
        ],
        out_shape=[
            jax.ShapeDtypeStruct((m, c), _F32),
            jax.ShapeDtypeStruct((m, c), _F32),
        ],
    )(x2d, wp, wq, b1r)


def _sc_gather_call(table, gidx):
    r, d = table.shape
    (m,) = gidx.shape
    try:
        info = plsc.get_sparse_core_info()
        nc, ns = info.num_cores, info.num_subcores
    except Exception:
        nc, ns = 2, 16
    nw = nc * ns
    ch = 128
    nb = 4
    per_w = m // nw
    nchunk = per_w // ch
    nouter = nchunk // nb
    mesh = plsc.VectorSubcoreMesh(core_axis_name="c", subcore_axis_name="s")

    scratch = ([pltpu.VMEM((ch,), jnp.int32) for _ in range(nb)]
               + [pltpu.VMEM((ch, d), _F32) for _ in range(nb)]
               + [pltpu.SemaphoreType.DMA for _ in range(3 * nb)])

    @functools.partial(
        pl.kernel,
        out_type=jax.ShapeDtypeStruct((m, d), _F32),
        mesh=mesh,
        scratch_types=scratch,
    )
    def k(table_hbm, idx_hbm, out_hbm, *scr):
        idx_v = scr[:nb]
        rows_v = scr[nb:2 * nb]
        sem_i = scr[2 * nb:3 * nb]
        sem_g = scr[3 * nb:4 * nb]
        sem_o = scr[4 * nb:5 * nb]
        wid = jax.lax.axis_index("s") * nc + jax.lax.axis_index("c")
        base0 = wid * per_w

        def idx_cp(i, b):
            return pltpu.make_async_copy(
                idx_hbm.at[pl.ds(base0 + i * ch, ch)], idx_v[b], sem_i[b])

        def gat_cp(b):
            return pltpu.make_async_copy(table_hbm.at[idx_v[b]], rows_v[b],
                                         sem_g[b])

        def out_cp(i, b):
            return pltpu.make_async_copy(
                rows_v[b], out_hbm.at[pl.ds(base0 + i * ch, ch)], sem_o[b])

        for b in range(nb):
            idx_cp(b, b).start()
        for b in range(2):
            idx_cp(b, b).wait()
            gat_cp(b).start()

        def outer(o, carry):
            for b in range(nb):
                i = o * nb + b
                gat_cp(b).wait()
                out_cp(i, b).start()
                jx = i + nb

                @pl.when(jx < nchunk)
                def _():
                    idx_cp(jx, b).start()

                j = i + 2
                bj = (b + 2) % nb

                @pl.when(j < nchunk)
                def _():
                    idx_cp(j, bj).wait()

                    @pl.when(j >= nb)
                    def _():
                        out_cp(j - nb, bj).wait()

                    gat_cp(bj).start()
            return carry

        jax.lax.fori_loop(0, nouter, outer, 0)
        for b in range(nb):
            out_cp(nchunk - nb + b, b).wait()

    return k(table, gidx)


def _edge_body(p_ref, qg_ref, g_ref, bb_ref, out_ref):
    p = p_ref[0]
    qg = qg_ref[0]
    h = jnp.maximum(p[:, None, :] + qg, 0.0)
    mu = jnp.mean(h, axis=-1, keepdims=True)
    var = jnp.mean((h - mu) ** 2, axis=-1, keepdims=True)
    ln = (h - mu) / jnp.sqrt(var + 1e-5) * g_ref[0][None, :] + bb_ref[0][None, :]
    out_ref[0] = jnp.max(ln, axis=1)


def _edge_call(p3, qg, ln1_g, ln1_b, br):
    b, n, k, c = qg.shape
    grid = (b, n // br)
    return pl.pallas_call(
        _edge_body,
        grid=grid,
        in_specs=[
            pl.BlockSpec((1, br, c), lambda bi, i: (bi, i, 0)),
            pl.BlockSpec((1, br, k, c), lambda bi, i: (bi, i, 0, 0)),
            pl.BlockSpec((1, c), lambda bi, i: (0, 0)),
            pl.BlockSpec((1, c), lambda bi, i: (0, 0)),
        ],
        out_specs=pl.BlockSpec((1, br, c), lambda bi, i: (bi, i, 0)),
        out_shape=jax.ShapeDtypeStruct((b, n, c), _F32),
    )(p3, qg, ln1_g.reshape(1, c), ln1_b.reshape(1, c))


def _pool_body(loc_ref, m_ref, g_ref):
    loc = loc_ref[0]
    msk = m_ref[0]
    masked = jnp.where(msk == 0.0, -1000000000.0, loc)
    g_ref[0, 0] = jnp.max(masked, axis=0)


def _pool_call(local, mask):
    b, n, c = local.shape
    return pl.pallas_call(
        _pool_body,
        grid=(b,),
        in_specs=[
            pl.BlockSpec((1, n, c), lambda bi: (bi, 0, 0)),
            pl.BlockSpec((1, n, 1), lambda bi: (bi, 0, 0)),
        ],
        out_specs=pl.BlockSpec((1, 1, c), lambda bi: (bi, 0, 0)),
        out_shape=jax.ShapeDtypeStruct((b, 1, c), _F32),
    )(local, mask[:, :, None]).reshape(b, c)


def _head_body(g_ref, w_ref, b2_ref, lg_ref, lb_ref, out_ref):
    o = _dot(g_ref[...], w_ref[...], ((1,), (0,))) + b2_ref[...]
    mu = jnp.mean(o, axis=-1, keepdims=True)
    var = jnp.mean((o - mu) ** 2, axis=-1, keepdims=True)
    out_ref[...] = (o - mu) / jnp.sqrt(var + 1e-5) * lg_ref[...] + lb_ref[...]


def _head_call(g, w2t, b2, ln2_g, ln2_b):
    b, c = g.shape
    return pl.pallas_call(
        _head_body,
        in_specs=[pl.BlockSpec((b, c), lambda: (0, 0)),
                  pl.BlockSpec((c, c), lambda: (0, 0)),
                  pl.BlockSpec((1, c), lambda: (0, 0)),
                  pl.BlockSpec((1, c), lambda: (0, 0)),
                  pl.BlockSpec((1, c), lambda: (0, 0))],
        out_specs=pl.BlockSpec((b, c), lambda: (0, 0)),
        out_shape=jax.ShapeDtypeStruct((b, c), _F32),
    )(g, w2t, b2.reshape(1, c), ln2_g.reshape(1, c), ln2_b.reshape(1, c))


def kernel(x, coords, mask, W1, b1, ln1_g, ln1_b, W2, b2, ln2_g, ln2_b):
    b, n, c = x.shape
    k = 16
    br = 256 if n % 256 == 0 else n

    gidx = _knn_call(x, br, k)

    wq = jnp.transpose(W1[:, c:])
    wp = jnp.transpose(W1[:, :c] - W1[:, c:])
    p2, q2 = _pq_call(x.reshape(b * n, c), wp, wq, b1.reshape(1, c), 512)
    p3 = p2.reshape(b, n, c)

    qg2 = jnp.take(q2, gidx.reshape(-1), axis=0)
    qg = qg2.reshape(b, n, k, c)

    local = _edge_call(p3, qg, ln1_g, ln1_b, br)
    g = _pool_call(local, mask)
    return _head_call(g, jnp.transpose(W2), b2, ln2_g, ln2_b)
